# baseline (speedup 1.0000x reference)
import functools

import jax
import jax.numpy as jnp
from jax import lax
from jax.experimental import pallas as pl
from jax.experimental.pallas import tpu as pltpu
from jax.experimental.pallas import tpu_sc as plsc

B = 16384
D = 32
NC = 2
NS = 16
NW = NC * NS
BPW = B // NW
CHUNK = 128
NCHUNK = BPW // CHUNK
L = 16
NGRP = BPW // L
BW = 16


def _sc_body(uid_hbm, iid_hbm, u_hbm, i_hbm, ub_hbm, ib_hbm,
             dots_hbm, bias_hbm,
             uidx, iidx, urow, irow, urows, irows, dots_v, ubr, ibr, bsum, sem):
    wid = lax.axis_index("s") * NC + lax.axis_index("c")
    base = wid * BPW
    pltpu.sync_copy(uid_hbm.at[pl.ds(base, BPW)], uidx)
    pltpu.sync_copy(iid_hbm.at[pl.ds(base, BPW)], iidx)

    def rows_body(g, carry):
        sl = pl.ds(g * L, L)
        urow[sl] = lax.shift_right_logical(uidx[sl], 4)
        irow[sl] = lax.shift_right_logical(iidx[sl], 4)
        return carry

    lax.fori_loop(0, NGRP, rows_body, 0)

    copies = []
    for c in range(NCHUNK):
        sl = pl.ds(c * CHUNK, CHUNK)
        copies.append(pltpu.async_copy(u_hbm.at[uidx.at[sl]], urows.at[sl], sem))
        copies.append(pltpu.async_copy(i_hbm.at[iidx.at[sl]], irows.at[sl], sem))
        copies.append(pltpu.async_copy(ub_hbm.at[urow.at[sl]], ubr.at[sl], sem))
        copies.append(pltpu.async_copy(ib_hbm.at[irow.at[sl]], ibr.at[sl], sem))
    for cp in copies:
        cp.wait()

    def dots_body(j, carry):
        for h in range(D // L):
            sl = pl.ds(h * L, L)
            dots_v[j, sl] = urows[j, sl] * irows[j, sl]
        return carry

    lax.fori_loop(0, BPW, dots_body, 0)

    def bias_body(g, carry):
        sl = pl.ds(g * L, L)
        pos = lax.iota(jnp.int32, L) + g * L
        ulane = lax.bitwise_and(uidx[sl], 15)
        ilane = lax.bitwise_and(iidx[sl], 15)
        ubv = plsc.load_gather(ubr, [pos, ulane])
        ibv = plsc.load_gather(ibr, [pos, ilane])
        bsum[sl] = ubv + ibv
        return carry

    lax.fori_loop(0, NGRP, bias_body, 0)

    pltpu.sync_copy(dots_v, dots_hbm.at[pl.ds(base, BPW)])
    pltpu.sync_copy(bsum, bias_hbm.at[pl.ds(base, BPW)])


@functools.cache
def _sc_gather():
    mesh = plsc.VectorSubcoreMesh(
        core_axis_name="c", subcore_axis_name="s", num_cores=NC, num_subcores=NS
    )
    return pl.kernel(
        _sc_body,
        out_type=(
            jax.ShapeDtypeStruct((B, D), jnp.float32),
            jax.ShapeDtypeStruct((B,), jnp.float32),
        ),
        mesh=mesh,
        compiler_params=pltpu.CompilerParams(
            use_tc_tiling_on_sc=False, needs_layout_passes=False
        ),
        scratch_types=(
            pltpu.VMEM((BPW,), jnp.int32),
            pltpu.VMEM((BPW,), jnp.int32),
            pltpu.VMEM((BPW,), jnp.int32),
            pltpu.VMEM((BPW,), jnp.int32),
            pltpu.VMEM((BPW, D), jnp.float32),
            pltpu.VMEM((BPW, D), jnp.float32),
            pltpu.VMEM((BPW, D), jnp.float32),
            pltpu.VMEM((BPW, BW), jnp.float32),
            pltpu.VMEM((BPW, BW), jnp.float32),
            pltpu.VMEM((BPW,), jnp.float32),
            pltpu.SemaphoreType.DMA,
        ),
    )


def _tc_body(x_ref, bias_ref, w1t_ref, b1_ref, w2t_ref, b2_ref, o_ref):
    x = x_ref[...]
    h = jax.nn.sigmoid(
        jnp.dot(x, w1t_ref[...], preferred_element_type=jnp.float32) + b1_ref[...]
    )
    z = jnp.dot(h, w2t_ref[...], preferred_element_type=jnp.float32) + b2_ref[...]
    res = 1.0 + 0.5 * (jax.nn.sigmoid(z) - 0.5)
    o_ref[...] = jnp.sum(x * res, axis=1) + bias_ref[...]


_tc_mlp = pl.pallas_call(
    _tc_body,
    out_shape=jax.ShapeDtypeStruct((B,), jnp.float32),
)


def kernel(user_ids, item_ids, U, I, W1, b1, W2, b2, Ub, Ib):
    ub16 = Ub.reshape(-1, BW)
    ib16 = Ib.reshape(-1, BW)
    dots, bias = _sc_gather()(user_ids, item_ids, U, I, ub16, ib16)
    return _tc_mlp(dots, bias, W1.T, b1.reshape(1, -1), W2.T, b2.reshape(1, -1))
